# baseline (speedup 1.0000x reference)
import functools

import jax
import jax.numpy as jnp
from jax import lax
from jax.experimental import pallas as pl
from jax.experimental.pallas import tpu as pltpu
from jax.experimental.pallas import tpu_sc as plsc

B, C, H, W = 2, 96, 512, 512
HW = H * W
N = B * HW
CP = 128

NC, NS, L = 2, 16, 16
NW = NC * NS
PIX_PER_W = N // NW
K = 128
CHUNKS = PIX_PER_W // K

_HB = 128


def _prep_body(in_ref, idx_ref, w_ref):
    b = pl.program_id(0)
    h = pl.program_id(1)
    d = in_ref[0]
    dx = d[0]
    dy = d[1]
    gy = lax.broadcasted_iota(jnp.int32, (_HB, W), 0).astype(jnp.float32) + (
        h * _HB).astype(jnp.float32)
    gx = lax.broadcasted_iota(jnp.int32, (_HB, W), 1).astype(jnp.float32)
    xf = gx + dx
    yf = gy + dy
    x0 = jnp.floor(xf)
    y0 = jnp.floor(yf)
    x1 = x0 + 1.0
    y1 = y0 + 1.0
    wx1 = xf - x0
    wx0 = 1.0 - wx1
    wy1 = yf - y0
    wy0 = 1.0 - wy1
    fW = jnp.float32(W - 1)
    fH = jnp.float32(H - 1)
    vx0 = ((x0 >= 0) & (x0 <= fW)).astype(jnp.float32)
    vx1 = ((x1 >= 0) & (x1 <= fW)).astype(jnp.float32)
    vy0 = ((y0 >= 0) & (y0 <= fH)).astype(jnp.float32)
    vy1 = ((y1 >= 0) & (y1 <= fH)).astype(jnp.float32)
    x0c = jnp.clip(x0, 0.0, fW).astype(jnp.int32)
    x1c = jnp.clip(x1, 0.0, fW).astype(jnp.int32)
    y0c = jnp.clip(y0, 0.0, fH).astype(jnp.int32)
    y1c = jnp.clip(y1, 0.0, fH).astype(jnp.int32)
    base = b * HW
    idx_ref[0, 0] = base + y0c * W + x0c
    idx_ref[1, 0] = base + y0c * W + x1c
    idx_ref[2, 0] = base + y1c * W + x0c
    idx_ref[3, 0] = base + y1c * W + x1c
    w_ref[0, 0] = wx0 * wy0 * vx0 * vy0
    w_ref[1, 0] = wx1 * wy0 * vx1 * vy0
    w_ref[2, 0] = wx0 * wy1 * vx0 * vy1
    w_ref[3, 0] = wx1 * wy1 * vx1 * vy1


def _prep(input2, interpret=False):
    return pl.pallas_call(
        _prep_body,
        grid=(B, H // _HB),
        in_specs=[pl.BlockSpec((1, 2, _HB, W), lambda b, h: (b, 0, h, 0))],
        out_specs=[
            pl.BlockSpec((4, 1, _HB, W), lambda b, h: (0, b, h, 0)),
            pl.BlockSpec((4, 1, _HB, W), lambda b, h: (0, b, h, 0)),
        ],
        out_shape=[
            jax.ShapeDtypeStruct((4, B, H, W), jnp.int32),
            jax.ShapeDtypeStruct((4, B, H, W), jnp.float32),
        ],
        interpret=interpret,
    )(input2)


def _sc_warp_body(table, idx4, w4, out, idx_v, w_v, rows_v, ob_v, sem):
    wid = lax.axis_index("s") * NC + lax.axis_index("c")
    pix0 = wid * PIX_PER_W

    def chunk(g, _):
        base = pix0 + g * K
        pltpu.sync_copy(idx4.at[:, pl.ds(base, K)], idx_v)
        pltpu.sync_copy(w4.at[:, pl.ds(base, K)], w_v)
        descs = [
            pltpu.async_copy(table.at[idx_v.at[i]], rows_v.at[i], sem)
            for i in range(4)
        ]
        for d in descs:
            d.wait()

        def xg_body(xg, _):
            p16 = lax.broadcasted_iota(jnp.int32, (L,), 0) + xg * L
            w0 = w_v[0, pl.ds(xg * L, L)]
            w1 = w_v[1, pl.ds(xg * L, L)]
            w2 = w_v[2, pl.ds(xg * L, L)]
            w3 = w_v[3, pl.ds(xg * L, L)]

            def c_body(cb, _):
                for cu in range(8):
                    c = cb * 8 + cu
                    cs = jnp.full((L,), 0, jnp.int32) + c
                    v0 = plsc.load_gather(rows_v.at[0], [p16, cs])
                    v1 = plsc.load_gather(rows_v.at[1], [p16, cs])
                    v2 = plsc.load_gather(rows_v.at[2], [p16, cs])
                    v3 = plsc.load_gather(rows_v.at[3], [p16, cs])
                    acc = v0 * w0 + v1 * w1 + v2 * w2 + v3 * w3
                    ob_v[c, pl.ds(xg * L, L)] = acc
                return 0

            lax.fori_loop(0, C // 8, c_body, 0)
            return 0

        lax.fori_loop(0, K // L, xg_body, 0)

        bb = base // HW
        rem = base - bb * HW
        yy = rem // W
        xx = rem - yy * W
        pltpu.sync_copy(ob_v, out.at[bb, :, yy, pl.ds(xx, K)])
        return 0

    lax.fori_loop(0, CHUNKS, chunk, 0)


@functools.lru_cache(maxsize=1)
def _sc_warp():
    return pl.kernel(
        _sc_warp_body,
        out_type=jax.ShapeDtypeStruct((B, C, H, W), jnp.float32),
        mesh=plsc.VectorSubcoreMesh(core_axis_name="c", subcore_axis_name="s"),
        compiler_params=pltpu.CompilerParams(needs_layout_passes=False),
        scratch_types=[
            pltpu.VMEM((4, K), jnp.int32),
            pltpu.VMEM((4, K), jnp.float32),
            pltpu.VMEM((4, K, CP), jnp.float32),
            pltpu.VMEM((C, K), jnp.float32),
            pltpu.SemaphoreType.DMA,
        ],
    )


def kernel(input1, input2):
    if input2.shape[1] == 3:
        input2 = input2[:, :2, :, :]
    table = jnp.transpose(input1, (0, 2, 3, 1)).reshape(N, C)
    table = jnp.pad(table, ((0, 0), (0, CP - C)))
    idx4, w4 = _prep(input2)
    idx4 = idx4.reshape(4, N)
    w4 = w4.reshape(4, N)
    return _sc_warp()(table, idx4, w4)


# speedup vs baseline: 2.0465x; 2.0465x over previous
import functools

import jax
import jax.numpy as jnp
from jax import lax
from jax.experimental import pallas as pl
from jax.experimental.pallas import tpu as pltpu
from jax.experimental.pallas import tpu_sc as plsc

B, C, H, W = 2, 96, 512, 512
HW = H * W
N = B * HW
CP = 128

NC, NS, L = 2, 16, 16
NW = NC * NS
PIX_PER_W = N // NW
K = 128
CHUNKS = PIX_PER_W // K

_HB = 128
_SKIP_COMPUTE = False
_SKIP_GATHER = False


def _prep_body(in_ref, idx_ref, w_ref):
    b = pl.program_id(0)
    h = pl.program_id(1)
    d = in_ref[0]
    dx = d[0]
    dy = d[1]
    gy = lax.broadcasted_iota(jnp.int32, (_HB, W), 0).astype(jnp.float32) + (
        h * _HB).astype(jnp.float32)
    gx = lax.broadcasted_iota(jnp.int32, (_HB, W), 1).astype(jnp.float32)
    xf = gx + dx
    yf = gy + dy
    x0 = jnp.floor(xf)
    y0 = jnp.floor(yf)
    x1 = x0 + 1.0
    y1 = y0 + 1.0
    wx1 = xf - x0
    wx0 = 1.0 - wx1
    wy1 = yf - y0
    wy0 = 1.0 - wy1
    fW = jnp.float32(W - 1)
    fH = jnp.float32(H - 1)
    vx0 = ((x0 >= 0) & (x0 <= fW)).astype(jnp.float32)
    vx1 = ((x1 >= 0) & (x1 <= fW)).astype(jnp.float32)
    vy0 = ((y0 >= 0) & (y0 <= fH)).astype(jnp.float32)
    vy1 = ((y1 >= 0) & (y1 <= fH)).astype(jnp.float32)
    x0c = jnp.clip(x0, 0.0, fW).astype(jnp.int32)
    x1c = jnp.clip(x1, 0.0, fW).astype(jnp.int32)
    y0c = jnp.clip(y0, 0.0, fH).astype(jnp.int32)
    y1c = jnp.clip(y1, 0.0, fH).astype(jnp.int32)
    base = b * HW
    idx_ref[0, 0] = base + y0c * W + x0c
    idx_ref[1, 0] = base + y0c * W + x1c
    idx_ref[2, 0] = base + y1c * W + x0c
    idx_ref[3, 0] = base + y1c * W + x1c
    w_ref[0, 0] = wx0 * wy0 * vx0 * vy0
    w_ref[1, 0] = wx1 * wy0 * vx1 * vy0
    w_ref[2, 0] = wx0 * wy1 * vx0 * vy1
    w_ref[3, 0] = wx1 * wy1 * vx1 * vy1


def _prep(input2, interpret=False):
    return pl.pallas_call(
        _prep_body,
        grid=(B, H // _HB),
        in_specs=[pl.BlockSpec((1, 2, _HB, W), lambda b, h: (b, 0, h, 0))],
        out_specs=[
            pl.BlockSpec((4, 1, _HB, W), lambda b, h: (0, b, h, 0)),
            pl.BlockSpec((4, 1, _HB, W), lambda b, h: (0, b, h, 0)),
        ],
        out_shape=[
            jax.ShapeDtypeStruct((4, B, H, W), jnp.int32),
            jax.ShapeDtypeStruct((4, B, H, W), jnp.float32),
        ],
        interpret=interpret,
    )(input2)


def _sc_warp_body(table, idx4, w4, out, idx_v, w_v, rows_v, ob_v, sem):
    wid = lax.axis_index("s") * NC + lax.axis_index("c")
    pix0 = wid * PIX_PER_W

    def chunk(g, _):
        base = pix0 + g * K
        pltpu.sync_copy(idx4.at[:, pl.ds(base, K)], idx_v)
        pltpu.sync_copy(w4.at[:, pl.ds(base, K)], w_v)
        if not _SKIP_GATHER:
            descs = [
                pltpu.async_copy(table.at[idx_v.at[i]], rows_v.at[i], sem)
                for i in range(4)
            ]
            for d in descs:
                d.wait()

        def px_body(p, _):
            zz = jnp.full((L,), 0, jnp.int32)
            pp = zz + p
            w0 = plsc.load_gather(w_v, [zz, pp])
            w1 = plsc.load_gather(w_v, [zz + 1, pp])
            w2 = plsc.load_gather(w_v, [zz + 2, pp])
            w3 = plsc.load_gather(w_v, [zz + 3, pp])
            ci = lax.broadcasted_iota(jnp.int32, (L,), 0)
            for cb in range(C // L):
                co = cb * L
                v0 = rows_v[0, p, pl.ds(co, L)]
                v1 = rows_v[1, p, pl.ds(co, L)]
                v2 = rows_v[2, p, pl.ds(co, L)]
                v3 = rows_v[3, p, pl.ds(co, L)]
                acc = v0 * w0 + v1 * w1 + v2 * w2 + v3 * w3
                plsc.store_scatter(ob_v, [ci + co, pp], acc)
            return 0

        if not _SKIP_COMPUTE:
            lax.fori_loop(0, K, px_body, 0)

        bb = base // HW
        rem = base - bb * HW
        yy = rem // W
        xx = rem - yy * W
        pltpu.sync_copy(ob_v.at[:, pl.ds(0, K)], out.at[bb, :, yy, pl.ds(xx, K)])
        return 0

    lax.fori_loop(0, CHUNKS, chunk, 0)


@functools.lru_cache(maxsize=1)
def _sc_warp():
    return pl.kernel(
        _sc_warp_body,
        out_type=jax.ShapeDtypeStruct((B, C, H, W), jnp.float32),
        mesh=plsc.VectorSubcoreMesh(core_axis_name="c", subcore_axis_name="s"),
        compiler_params=pltpu.CompilerParams(needs_layout_passes=False),
        scratch_types=[
            pltpu.VMEM((4, K), jnp.int32),
            pltpu.VMEM((4, K), jnp.float32),
            pltpu.VMEM((4, K, CP), jnp.float32),
            pltpu.VMEM((C, K + 1), jnp.float32),
            pltpu.SemaphoreType.DMA,
        ],
    )


def kernel(input1, input2):
    if input2.shape[1] == 3:
        input2 = input2[:, :2, :, :]
    table = jnp.transpose(input1, (0, 2, 3, 1)).reshape(N, C)
    table = jnp.pad(table, ((0, 0), (0, CP - C)))
    idx4, w4 = _prep(input2)
    idx4 = idx4.reshape(4, N)
    w4 = w4.reshape(4, N)
    return _sc_warp()(table, idx4, w4)


# speedup vs baseline: 2.0807x; 1.0167x over previous
import functools

import jax
import jax.numpy as jnp
from jax import lax
from jax.experimental import pallas as pl
from jax.experimental.pallas import tpu as pltpu
from jax.experimental.pallas import tpu_sc as plsc

B, C, H, W = 2, 96, 512, 512
HW = H * W
N = B * HW
CP = 128

NC, NS, L = 2, 16, 16
NW = NC * NS
PIX_PER_W = N // NW
K = 128
CHUNKS = PIX_PER_W // K

_HB = 128
_SKIP_COMPUTE = False
_SKIP_GATHER = False


_BCAST_DNUMS = lax.GatherDimensionNumbers(
    offset_dims=(), collapsed_slice_dims=(0,), start_index_map=(0,))


def _lane_bcast(vec, j):
    idx = jnp.full((L, 1), j, jnp.int32)
    return lax.gather(vec, idx, _BCAST_DNUMS, (1,),
                      mode=lax.GatherScatterMode.PROMISE_IN_BOUNDS)


def _prep_body(in_ref, idx_ref, w_ref):
    b = pl.program_id(0)
    h = pl.program_id(1)
    d = in_ref[0]
    dx = d[0]
    dy = d[1]
    gy = lax.broadcasted_iota(jnp.int32, (_HB, W), 0).astype(jnp.float32) + (
        h * _HB).astype(jnp.float32)
    gx = lax.broadcasted_iota(jnp.int32, (_HB, W), 1).astype(jnp.float32)
    xf = gx + dx
    yf = gy + dy
    x0 = jnp.floor(xf)
    y0 = jnp.floor(yf)
    x1 = x0 + 1.0
    y1 = y0 + 1.0
    wx1 = xf - x0
    wx0 = 1.0 - wx1
    wy1 = yf - y0
    wy0 = 1.0 - wy1
    fW = jnp.float32(W - 1)
    fH = jnp.float32(H - 1)
    vx0 = ((x0 >= 0) & (x0 <= fW)).astype(jnp.float32)
    vx1 = ((x1 >= 0) & (x1 <= fW)).astype(jnp.float32)
    vy0 = ((y0 >= 0) & (y0 <= fH)).astype(jnp.float32)
    vy1 = ((y1 >= 0) & (y1 <= fH)).astype(jnp.float32)
    x0c = jnp.clip(x0, 0.0, fW).astype(jnp.int32)
    x1c = jnp.clip(x1, 0.0, fW).astype(jnp.int32)
    y0c = jnp.clip(y0, 0.0, fH).astype(jnp.int32)
    y1c = jnp.clip(y1, 0.0, fH).astype(jnp.int32)
    base = b * HW
    idx_ref[0, 0] = base + y0c * W + x0c
    idx_ref[1, 0] = base + y0c * W + x1c
    idx_ref[2, 0] = base + y1c * W + x0c
    idx_ref[3, 0] = base + y1c * W + x1c
    w_ref[0, 0] = wx0 * wy0 * vx0 * vy0
    w_ref[1, 0] = wx1 * wy0 * vx1 * vy0
    w_ref[2, 0] = wx0 * wy1 * vx0 * vy1
    w_ref[3, 0] = wx1 * wy1 * vx1 * vy1


def _prep(input2, interpret=False):
    return pl.pallas_call(
        _prep_body,
        grid=(B, H // _HB),
        in_specs=[pl.BlockSpec((1, 2, _HB, W), lambda b, h: (b, 0, h, 0))],
        out_specs=[
            pl.BlockSpec((4, 1, _HB, W), lambda b, h: (0, b, h, 0)),
            pl.BlockSpec((4, 1, _HB, W), lambda b, h: (0, b, h, 0)),
        ],
        out_shape=[
            jax.ShapeDtypeStruct((4, B, H, W), jnp.int32),
            jax.ShapeDtypeStruct((4, B, H, W), jnp.float32),
        ],
        interpret=interpret,
    )(input2)


def _sc_warp_body(table, idx4, w4, out, idx_v, w_v, rows_v, ob_v, sem):
    wid = lax.axis_index("s") * NC + lax.axis_index("c")
    pix0 = wid * PIX_PER_W

    def chunk(g, _):
        base = pix0 + g * K
        pltpu.sync_copy(idx4.at[:, pl.ds(base, K)], idx_v)
        pltpu.sync_copy(w4.at[:, pl.ds(base, K)], w_v)
        if not _SKIP_GATHER:
            descs = [
                pltpu.async_copy(table.at[idx_v.at[i]], rows_v.at[i], sem)
                for i in range(4)
            ]
            for d in descs:
                d.wait()

        def xg_body(xg, _):
            zz = jnp.full((L,), 0, jnp.int32)
            ci = lax.broadcasted_iota(jnp.int32, (L,), 0)
            w16 = [w_v[i, pl.ds(xg * L, L)] for i in range(4)]
            pbase = xg * L
            for j in range(L):
                wj = [_lane_bcast(w16[i], j) for i in range(4)]
                p = pbase + j
                pp = zz + p
                for cb in range(C // L):
                    co = cb * L
                    v0 = rows_v[0, p, pl.ds(co, L)]
                    v1 = rows_v[1, p, pl.ds(co, L)]
                    v2 = rows_v[2, p, pl.ds(co, L)]
                    v3 = rows_v[3, p, pl.ds(co, L)]
                    acc = v0 * wj[0] + v1 * wj[1] + v2 * wj[2] + v3 * wj[3]
                    plsc.store_scatter(ob_v, [ci + co, pp], acc)
            return 0

        if not _SKIP_COMPUTE:
            lax.fori_loop(0, K // L, xg_body, 0)

        bb = base // HW
        rem = base - bb * HW
        yy = rem // W
        xx = rem - yy * W
        pltpu.sync_copy(ob_v.at[:, pl.ds(0, K)], out.at[bb, :, yy, pl.ds(xx, K)])
        return 0

    lax.fori_loop(0, CHUNKS, chunk, 0)


@functools.lru_cache(maxsize=1)
def _sc_warp():
    return pl.kernel(
        _sc_warp_body,
        out_type=jax.ShapeDtypeStruct((B, C, H, W), jnp.float32),
        mesh=plsc.VectorSubcoreMesh(core_axis_name="c", subcore_axis_name="s"),
        compiler_params=pltpu.CompilerParams(needs_layout_passes=False),
        scratch_types=[
            pltpu.VMEM((4, K), jnp.int32),
            pltpu.VMEM((4, K), jnp.float32),
            pltpu.VMEM((4, K, CP), jnp.float32),
            pltpu.VMEM((C, K + 1), jnp.float32),
            pltpu.SemaphoreType.DMA,
        ],
    )


def kernel(input1, input2):
    if input2.shape[1] == 3:
        input2 = input2[:, :2, :, :]
    table = jnp.transpose(input1, (0, 2, 3, 1)).reshape(N, C)
    table = jnp.pad(table, ((0, 0), (0, CP - C)))
    idx4, w4 = _prep(input2)
    idx4 = idx4.reshape(4, N)
    w4 = w4.reshape(4, N)
    return _sc_warp()(table, idx4, w4)


# speedup vs baseline: 2.4032x; 1.1550x over previous
import functools

import jax
import jax.numpy as jnp
from jax import lax
from jax.experimental import pallas as pl
from jax.experimental.pallas import tpu as pltpu
from jax.experimental.pallas import tpu_sc as plsc

B, C, H, W = 2, 96, 512, 512
HW = H * W
N = B * HW
CP = 128

NC, NS, L = 2, 16, 16
NW = NC * NS
PIX_PER_W = N // NW
K = 128
CHUNKS = PIX_PER_W // K

_HB = 128
_SKIP_COMPUTE = False
_SKIP_GATHER = False


_BCAST_DNUMS = lax.GatherDimensionNumbers(
    offset_dims=(), collapsed_slice_dims=(0,), start_index_map=(0,))


def _lane_bcast(vec, j):
    idx = jnp.full((L, 1), j, jnp.int32)
    return lax.gather(vec, idx, _BCAST_DNUMS, (1,),
                      mode=lax.GatherScatterMode.PROMISE_IN_BOUNDS)


def _prep_body(in_ref, idx_ref, w_ref):
    b = pl.program_id(0)
    h = pl.program_id(1)
    d = in_ref[0]
    dx = d[0]
    dy = d[1]
    gy = lax.broadcasted_iota(jnp.int32, (_HB, W), 0).astype(jnp.float32) + (
        h * _HB).astype(jnp.float32)
    gx = lax.broadcasted_iota(jnp.int32, (_HB, W), 1).astype(jnp.float32)
    xf = gx + dx
    yf = gy + dy
    x0 = jnp.floor(xf)
    y0 = jnp.floor(yf)
    x1 = x0 + 1.0
    y1 = y0 + 1.0
    wx1 = xf - x0
    wx0 = 1.0 - wx1
    wy1 = yf - y0
    wy0 = 1.0 - wy1
    fW = jnp.float32(W - 1)
    fH = jnp.float32(H - 1)
    vx0 = ((x0 >= 0) & (x0 <= fW)).astype(jnp.float32)
    vx1 = ((x1 >= 0) & (x1 <= fW)).astype(jnp.float32)
    vy0 = ((y0 >= 0) & (y0 <= fH)).astype(jnp.float32)
    vy1 = ((y1 >= 0) & (y1 <= fH)).astype(jnp.float32)
    x0c = jnp.clip(x0, 0.0, fW).astype(jnp.int32)
    x1c = jnp.clip(x1, 0.0, fW).astype(jnp.int32)
    y0c = jnp.clip(y0, 0.0, fH).astype(jnp.int32)
    y1c = jnp.clip(y1, 0.0, fH).astype(jnp.int32)
    base = b * HW
    idx_ref[0, 0] = base + y0c * W + x0c
    idx_ref[1, 0] = base + y0c * W + x1c
    idx_ref[2, 0] = base + y1c * W + x0c
    idx_ref[3, 0] = base + y1c * W + x1c
    w_ref[0, 0] = wx0 * wy0 * vx0 * vy0
    w_ref[1, 0] = wx1 * wy0 * vx1 * vy0
    w_ref[2, 0] = wx0 * wy1 * vx0 * vy1
    w_ref[3, 0] = wx1 * wy1 * vx1 * vy1


def _prep(input2, interpret=False):
    return pl.pallas_call(
        _prep_body,
        grid=(B, H // _HB),
        in_specs=[pl.BlockSpec((1, 2, _HB, W), lambda b, h: (b, 0, h, 0))],
        out_specs=[
            pl.BlockSpec((4, 1, _HB, W), lambda b, h: (0, b, h, 0)),
            pl.BlockSpec((4, 1, _HB, W), lambda b, h: (0, b, h, 0)),
        ],
        out_shape=[
            jax.ShapeDtypeStruct((4, B, H, W), jnp.int32),
            jax.ShapeDtypeStruct((4, B, H, W), jnp.float32),
        ],
        interpret=interpret,
    )(input2)


def _sc_warp_body(table, idx4, w4, out, idx_v, w_v, rows_v, ob_v, sem):
    wid = lax.axis_index("s") * NC + lax.axis_index("c")
    pix0 = wid * PIX_PER_W

    def chunk(g, _):
        base = pix0 + g * K
        pltpu.sync_copy(idx4.at[:, pl.ds(base, K)], idx_v)
        pltpu.sync_copy(w4.at[:, pl.ds(base, K)], w_v)
        if not _SKIP_GATHER:
            descs = [
                pltpu.async_copy(table.at[idx_v.at[i]], rows_v.at[i], sem)
                for i in range(4)
            ]
            for d in descs:
                d.wait()

        def xg_body(xg, _):
            zz = jnp.full((L,), 0, jnp.int32)
            ci = lax.broadcasted_iota(jnp.int32, (L,), 0)
            w16 = [w_v[i, pl.ds(xg * L, L)] for i in range(4)]
            pbase = xg * L
            for j in range(L):
                wj = [_lane_bcast(w16[i], j) for i in range(4)]
                p = pbase + j
                pp = zz + p
                vs = [[rows_v[i, p, pl.ds(cb * L, L)] for i in range(4)]
                      for cb in range(C // L)]
                for cb in range(C // L):
                    v0, v1, v2, v3 = vs[cb]
                    acc = v0 * wj[0] + v1 * wj[1] + v2 * wj[2] + v3 * wj[3]
                    plsc.store_scatter(ob_v, [ci + cb * L, pp], acc)
            return 0

        if not _SKIP_COMPUTE:
            lax.fori_loop(0, K // L, xg_body, 0)

        bb = base // HW
        rem = base - bb * HW
        yy = rem // W
        xx = rem - yy * W
        pltpu.sync_copy(ob_v.at[:, pl.ds(0, K)], out.at[bb, :, yy, pl.ds(xx, K)])
        return 0

    lax.fori_loop(0, CHUNKS, chunk, 0)


@functools.lru_cache(maxsize=1)
def _sc_warp():
    return pl.kernel(
        _sc_warp_body,
        out_type=jax.ShapeDtypeStruct((B, C, H, W), jnp.float32),
        mesh=plsc.VectorSubcoreMesh(core_axis_name="c", subcore_axis_name="s"),
        compiler_params=pltpu.CompilerParams(needs_layout_passes=False),
        scratch_types=[
            pltpu.VMEM((4, K), jnp.int32),
            pltpu.VMEM((4, K), jnp.float32),
            pltpu.VMEM((4, K, CP), jnp.float32),
            pltpu.VMEM((C, K + 1), jnp.float32),
            pltpu.SemaphoreType.DMA,
        ],
    )


def kernel(input1, input2):
    if input2.shape[1] == 3:
        input2 = input2[:, :2, :, :]
    table = jnp.transpose(input1, (0, 2, 3, 1)).reshape(N, C)
    table = jnp.pad(table, ((0, 0), (0, CP - C)))
    idx4, w4 = _prep(input2)
    idx4 = idx4.reshape(4, N)
    w4 = w4.reshape(4, N)
    return _sc_warp()(table, idx4, w4)
